# baseline (speedup 1.0000x reference)
import functools

import jax
import jax.numpy as jnp
from jax import lax
from jax.experimental import pallas as pl
from jax.experimental.pallas import tpu as pltpu
from jax.experimental.pallas import tpu_sc as plsc

_D_MODEL = 4096
_N_EXP = 64
_TOP_K = 8
_ID_BIAS = 1.0
_N_TOKENS = 8192
_N_CHUNKS = 2
_NEG_INF = float("-inf")

_NUM_CORES = 2
_NUM_SUBCORES = 16
_LANES = 16
_NW = _NUM_CORES * _NUM_SUBCORES


def _proj_block(h_ref, w_ref, out_ref):
    logits = lax.dot_general(
        w_ref[...],
        h_ref[...],
        dimension_numbers=(((1,), (1,)), ((), ())),
        preferred_element_type=jnp.float32,
    )
    idx_col = lax.broadcasted_iota(jnp.int32, (_N_EXP, 128), 0)[:, :1]
    out_ref[...] = logits + jnp.where(idx_col == _N_EXP - 1, _ID_BIAS, 0.0)


@functools.lru_cache(maxsize=None)
def _make_proj(n_tokens):
    block = min(1024, n_tokens)

    def proj(h, W):
        return pl.pallas_call(
            _proj_block,
            grid=(n_tokens // block,),
            in_specs=[
                pl.BlockSpec((block, _D_MODEL), lambda i: (i, 0)),
                pl.BlockSpec((_N_EXP, _D_MODEL), lambda i: (0, 0)),
            ],
            out_specs=pl.BlockSpec((_N_EXP, block), lambda i: (0, i)),
            out_shape=jax.ShapeDtypeStruct((_N_EXP, n_tokens), jnp.float32),
        )(h, W)

    return proj


_sc_mesh = plsc.VectorSubcoreMesh(
    core_axis_name="c", subcore_axis_name="s"
)


@functools.lru_cache(maxsize=None)
def _make_topk_sc(n_tokens):
    tpw = n_tokens // _NW
    groups = tpw // _LANES

    @functools.partial(
        pl.kernel,
        out_type=jax.ShapeDtypeStruct((_N_EXP, n_tokens), jnp.float32),
        mesh=_sc_mesh,
        scratch_types=[
            pltpu.VMEM((_N_EXP, tpw), jnp.float32),
            pltpu.VMEM((_N_EXP, tpw), jnp.float32),
        ],
    )
    def topk_sc(logits_hbm, gate_hbm, chunk_v, out_v):
        wid = lax.axis_index("s") * _NUM_CORES + lax.axis_index("c")
        base = wid * tpw
        pltpu.sync_copy(logits_hbm.at[:, pl.ds(base, tpw)], chunk_v)

        @plsc.parallel_loop(0, groups)
        def group(g):
            off = g * _LANES
            m = [jnp.full((_LANES,), _NEG_INF, jnp.float32)] * _TOP_K
            for e in range(_N_EXP):
                v = chunk_v[e, pl.ds(off, _LANES)]
                for r in range(_TOP_K):
                    hi = jnp.maximum(m[r], v)
                    v = jnp.minimum(m[r], v)
                    m[r] = hi
            thr = m[_TOP_K - 1]
            cgt = jnp.zeros((_LANES,), jnp.float32)
            for r in range(_TOP_K - 1):
                cgt = cgt + jnp.where(m[r] > thr, 1.0, 0.0)
            need = float(_TOP_K) - cgt
            run = jnp.zeros((_LANES,), jnp.float32)
            for e in range(_N_EXP):
                v = chunk_v[e, pl.ds(off, _LANES)]
                gt_f = jnp.where(v > thr, 1.0, 0.0)
                eq_f = jnp.where(v == thr, 1.0, 0.0)
                ok_f = jnp.where(run < need, eq_f, 0.0)
                run = run + eq_f
                out_v[e, pl.ds(off, _LANES)] = (1.0 / _TOP_K) * (gt_f + ok_f)

        pltpu.sync_copy(out_v, gate_hbm.at[:, pl.ds(base, tpw)])

    return topk_sc


@jax.jit
def _router(h, W):
    rows = _N_TOKENS // _N_CHUNKS
    proj = _make_proj(rows)
    topk = _make_topk_sc(rows)
    gates = []
    for c in range(_N_CHUNKS):
        hc = lax.slice(h, (c * rows, 0), ((c + 1) * rows, _D_MODEL))
        gates.append(topk(proj(hc, W)))
    gate_t = jnp.concatenate(gates, axis=1) if _N_CHUNKS > 1 else gates[0]
    gate = gate_t.T
    mask = gate != 0.0
    return mask, gate


def kernel(h, W):
    return _router(h, W)


# speedup vs baseline: 2.0944x; 2.0944x over previous
import functools

import jax
import jax.numpy as jnp
from jax import lax
from jax.experimental import pallas as pl
from jax.experimental.pallas import tpu as pltpu
from jax.experimental.pallas import tpu_sc as plsc

_D_MODEL = 4096
_N_EXP = 64
_TOP_K = 8
_ID_BIAS = 1.0
_N_TOKENS = 8192
_N_CHUNKS = 2
_NEG_INF = float("-inf")

_NUM_CORES = 2
_NUM_SUBCORES = 16
_LANES = 16
_NW = _NUM_CORES * _NUM_SUBCORES


def _proj_block(h_ref, w_ref, out_ref):
    logits = lax.dot_general(
        w_ref[...],
        h_ref[...],
        dimension_numbers=(((1,), (1,)), ((), ())),
        preferred_element_type=jnp.float32,
    )
    idx_col = lax.broadcasted_iota(jnp.int32, (_N_EXP, 128), 0)[:, :1]
    out_ref[...] = logits + jnp.where(idx_col == _N_EXP - 1, _ID_BIAS, 0.0)


@functools.lru_cache(maxsize=None)
def _make_proj(n_tokens, chunk_idx):
    block = min(1024, n_tokens)
    first_block = chunk_idx * (n_tokens // block)

    def proj(h, W):
        return pl.pallas_call(
            _proj_block,
            grid=(n_tokens // block,),
            in_specs=[
                pl.BlockSpec((block, _D_MODEL),
                             lambda i: (first_block + i, 0)),
                pl.BlockSpec((_N_EXP, _D_MODEL), lambda i: (0, 0)),
            ],
            out_specs=pl.BlockSpec((_N_EXP, block), lambda i: (0, i)),
            out_shape=jax.ShapeDtypeStruct((_N_EXP, n_tokens), jnp.float32),
        )(h, W)

    return proj


_sc_mesh = plsc.VectorSubcoreMesh(
    core_axis_name="c", subcore_axis_name="s"
)


@functools.lru_cache(maxsize=None)
def _make_topk_sc(n_tokens):
    tpw = n_tokens // _NW
    groups = tpw // _LANES

    @functools.partial(
        pl.kernel,
        out_type=jax.ShapeDtypeStruct((_N_EXP, n_tokens), jnp.float32),
        mesh=_sc_mesh,
        scratch_types=[
            pltpu.VMEM((_N_EXP, tpw), jnp.float32),
            pltpu.VMEM((_N_EXP, tpw), jnp.float32),
        ],
    )
    def topk_sc(logits_hbm, gate_hbm, chunk_v, out_v):
        wid = lax.axis_index("s") * _NUM_CORES + lax.axis_index("c")
        base = wid * tpw
        pltpu.sync_copy(logits_hbm.at[:, pl.ds(base, tpw)], chunk_v)

        @plsc.parallel_loop(0, groups)
        def group(g):
            off = g * _LANES
            m = [jnp.full((_LANES,), _NEG_INF, jnp.float32)] * _TOP_K
            for e in range(_N_EXP):
                v = chunk_v[e, pl.ds(off, _LANES)]
                for r in range(_TOP_K):
                    hi = jnp.maximum(m[r], v)
                    v = jnp.minimum(m[r], v)
                    m[r] = hi
            thr = m[_TOP_K - 1]
            cgt = jnp.zeros((_LANES,), jnp.float32)
            for r in range(_TOP_K - 1):
                cgt = cgt + jnp.where(m[r] > thr, 1.0, 0.0)
            need = float(_TOP_K) - cgt
            run = jnp.zeros((_LANES,), jnp.float32)
            for e in range(_N_EXP):
                v = chunk_v[e, pl.ds(off, _LANES)]
                gt_f = jnp.where(v > thr, 1.0, 0.0)
                eq_f = jnp.where(v == thr, 1.0, 0.0)
                ok_f = jnp.where(run < need, eq_f, 0.0)
                run = run + eq_f
                out_v[e, pl.ds(off, _LANES)] = (1.0 / _TOP_K) * (gt_f + ok_f)

        pltpu.sync_copy(out_v, gate_hbm.at[:, pl.ds(base, tpw)])

    return topk_sc


@jax.jit
def _router(h, W):
    rows = _N_TOKENS // _N_CHUNKS
    topk = _make_topk_sc(rows)
    gates = []
    for c in range(_N_CHUNKS):
        gates.append(topk(_make_proj(rows, c)(h, W)))
    gate_t = jnp.concatenate(gates, axis=1) if _N_CHUNKS > 1 else gates[0]
    gate = gate_t.T
    mask = gate != 0.0
    return mask, gate


def kernel(h, W):
    return _router(h, W)


# speedup vs baseline: 2.8737x; 1.3721x over previous
import functools

import jax
import jax.numpy as jnp
from jax.experimental import pallas as pl

_D_MODEL = 4096
_N_EXP = 64
_TOP_K = 8
_ID_BIAS = 1.0
_NEG_INF = float("-inf")


def _router_block(h_ref, w_ref, sel_ref, gate_ref):
    logits = jax.lax.dot_general(
        w_ref[...],
        h_ref[...],
        dimension_numbers=(((1,), (1,)), ((), ())),
        preferred_element_type=jnp.float32,
    )
    idx_col = jax.lax.broadcasted_iota(jnp.int32, (_N_EXP, 128), 0)[:, :1]
    logits = logits + jnp.where(idx_col == _N_EXP - 1, _ID_BIAS, 0.0)

    work = logits
    vals = []
    cnts = []
    for _ in range(_TOP_K):
        m = jnp.max(work, axis=0, keepdims=True)
        eq = work == m
        vals.append(m)
        cnts.append(jnp.sum(eq.astype(jnp.float32), axis=0, keepdims=True))
        work = jnp.where(eq, _NEG_INF, work)

    cum = cnts[0]
    thr = vals[0]
    gt_cnt = jnp.zeros_like(cum)
    for j in range(1, _TOP_K):
        below = cum < _TOP_K
        thr = jnp.where(below, vals[j], thr)
        gt_cnt = jnp.where(below, cum, gt_cnt)
        cum = cum + cnts[j]

    eq_thr = logits == thr
    row_i = jax.lax.broadcasted_iota(jnp.int32, (_N_EXP, _N_EXP), 0)
    col_i = jax.lax.broadcasted_iota(jnp.int32, (_N_EXP, _N_EXP), 1)
    ltri = (row_i >= col_i).astype(jnp.float32)
    prefix = jax.lax.dot_general(
        ltri,
        eq_thr.astype(jnp.float32),
        dimension_numbers=(((1,), (0,)), ((), ())),
        preferred_element_type=jnp.float32,
    )
    need = _TOP_K - gt_cnt
    sel = jnp.where(
        (logits > thr) | (eq_thr & (prefix <= need)), 1.0, 0.0
    )

    sel_t = sel.T
    sel_ref[...] = sel_t != 0.0
    gate_ref[...] = sel_t * (1.0 / _TOP_K)


@functools.partial(jax.jit, static_argnames=("block_rows",))
def _router(h, W, block_rows=1024):
    n_rows = h.shape[0]
    grid = (n_rows // block_rows,)
    sel, gate = pl.pallas_call(
        _router_block,
        grid=grid,
        in_specs=[
            pl.BlockSpec((block_rows, _D_MODEL), lambda i: (i, 0)),
            pl.BlockSpec((_N_EXP, _D_MODEL), lambda i: (0, 0)),
        ],
        out_specs=[
            pl.BlockSpec((block_rows, _N_EXP), lambda i: (i, 0)),
            pl.BlockSpec((block_rows, _N_EXP), lambda i: (i, 0)),
        ],
        out_shape=[
            jax.ShapeDtypeStruct((n_rows, _N_EXP), jnp.bool_),
            jax.ShapeDtypeStruct((n_rows, _N_EXP), jnp.float32),
        ],
    )(h, W)
    return sel, gate


def kernel(h, W):
    mask, gate = _router(h, W)
    return mask, gate
